# baseline (speedup 1.0000x reference)
import jax
import jax.numpy as jnp
from jax import lax
from jax.experimental import pallas as pl
from jax.experimental.pallas import tpu as pltpu
from jax.experimental.pallas import tpu_sc as plsc

N_NODES = 10000
N_GRAPHS = 500
BN_EPS = 1e-5
BN_SCALE = (1.0 + BN_EPS) ** -0.5

NC = 2
NS = 16
NW = NC * NS
CH = 128

K_EDGE = 160
N_ACC = 10112


def _make_edge_scatter(d, n_acc):
  mesh = plsc.VectorSubcoreMesh(core_axis_name="c", subcore_axis_name="s")
  z_per = n_acc // NS
  assert K_EDGE % 4 == 0 and z_per % 8 == 0 and z_per <= 5 * CH

  def body(table_hbm, src_hbm, dst_hbm, out_hbm,
           src_v, dst_v, b0, b1, b2, b3, acc,
           g0, g1, g2, g3, s0, s1, s2, s3):
    c = lax.axis_index("c")
    s = lax.axis_index("s")
    bufs = (b0, b1, b2, b3)
    gsems = (g0, g1, g2, g3)
    ssems = (s0, s1, s2, s3)

    @pl.when(c == 0)
    def _run():
      zv = jnp.zeros((16,), jnp.float32)

      def zrow(r, carry):
        for q in range(d // 16):
          b0[r, pl.ds(16 * q, 16)] = zv
        return carry

      lax.fori_loop(0, CH, zrow, 0)
      base = s * z_per
      left = z_per
      off = 0
      while left > 0:
        n = min(left, CH)
        pltpu.sync_copy(b0.at[pl.ds(0, n)], acc.at[pl.ds(base + off, n)])
        off += n
        left -= n
      pltpu.sync_copy(src_hbm.at[s], src_v)
      pltpu.sync_copy(dst_hbm.at[s], dst_v)

    plsc.subcore_barrier()

    def gather(k, j):
      pltpu.async_copy(table_hbm.at[src_v.at[k]], bufs[j], gsems[j])

    def scatter(k, j):
      pltpu.async_copy(bufs[j], acc.at[dst_v.at[k]], ssems[j], add=True)

    def wait_gather(k, j):
      pltpu.make_async_copy(table_hbm.at[src_v.at[k]], bufs[j],
                            gsems[j]).wait()

    def wait_scatter(k, j):
      pltpu.make_async_copy(bufs[j], acc.at[dst_v.at[k]], ssems[j]).wait()

    @pl.when(c == 0)
    def _edges():
      gather(0, 0)
      gather(1, 1)

      def step(m, carry):
        base = 4 * m
        for j in range(4):
          k = base + j
          jn = (j + 2) % 4

          @pl.when(k - 2 >= 0)
          def _():
            wait_scatter(k - 2, jn)

          @pl.when(k + 2 < K_EDGE)
          def _():
            gather(k + 2, jn)

          wait_gather(k, j)
          scatter(k, j)
        return carry

      lax.fori_loop(0, K_EDGE // 4, step, 0)
      wait_scatter(K_EDGE - 2, 2)
      wait_scatter(K_EDGE - 1, 3)

    plsc.subcore_barrier()

    @pl.when(c == 0)
    def _out():
      pltpu.sync_copy(acc.at[pl.ds(s * z_per, z_per)],
                      out_hbm.at[0, pl.ds(s * z_per, z_per)])

  return pl.kernel(
      body,
      out_type=jax.ShapeDtypeStruct((1, n_acc, d), jnp.float32),
      mesh=mesh,
      compiler_params=pltpu.CompilerParams(use_tc_tiling_on_sc=False),
      scratch_types=[
          pltpu.VMEM((K_EDGE, CH), jnp.int32),
          pltpu.VMEM((K_EDGE, CH), jnp.int32),
          pltpu.VMEM((CH, d), jnp.float32),
          pltpu.VMEM((CH, d), jnp.float32),
          pltpu.VMEM((CH, d), jnp.float32),
          pltpu.VMEM((CH, d), jnp.float32),
          pltpu.VMEM_SHARED((n_acc, d), jnp.float32),
          pltpu.SemaphoreType.DMA,
          pltpu.SemaphoreType.DMA,
          pltpu.SemaphoreType.DMA,
          pltpu.SemaphoreType.DMA,
          pltpu.SemaphoreType.DMA,
          pltpu.SemaphoreType.DMA,
          pltpu.SemaphoreType.DMA,
          pltpu.SemaphoreType.DMA,
      ],
  )


def _make_hist(d, n_acc):
  mesh = plsc.VectorSubcoreMesh(core_axis_name="c", subcore_axis_name="s")
  z_per = n_acc // NS
  assert z_per % 8 == 0

  def body(ones_hbm, dst_hbm, zeros_hbm, out_hbm,
           dst_v, rows_v, acc, s0, s1, s2, s3):
    c = lax.axis_index("c")
    s = lax.axis_index("s")
    ssems = (s0, s1, s2, s3)

    @pl.when(c == 0)
    def _init():
      pltpu.sync_copy(zeros_hbm.at[pl.ds(s * z_per, z_per)],
                      acc.at[pl.ds(s * z_per, z_per)])
      pltpu.sync_copy(dst_hbm.at[s], dst_v)
      pltpu.sync_copy(ones_hbm, rows_v)

    plsc.subcore_barrier()

    def scatter(k, jj):
      pltpu.async_copy(rows_v, acc.at[dst_v.at[k]], ssems[jj], add=True)

    def wait_scatter(k, jj):
      pltpu.make_async_copy(rows_v, acc.at[dst_v.at[k]], ssems[jj]).wait()

    @pl.when(c == 0)
    def _edges():
      scatter(0, 0)
      scatter(1, 1)
      scatter(2, 2)

      def step(m, carry):
        base = 4 * m
        for j in range(4):
          k = base + j

          @pl.when(k + 3 < K_EDGE)
          def _():
            scatter(k + 3, (j + 3) % 4)

          wait_scatter(k, j)
        return carry

      lax.fori_loop(0, K_EDGE // 4, step, 0)

    plsc.subcore_barrier()

    @pl.when(c == 0)
    def _out():
      pltpu.sync_copy(acc.at[pl.ds(s * z_per, z_per)],
                      out_hbm.at[0, pl.ds(s * z_per, z_per)])

  return pl.kernel(
      body,
      out_type=jax.ShapeDtypeStruct((1, n_acc, d), jnp.float32),
      mesh=mesh,
      compiler_params=pltpu.CompilerParams(use_tc_tiling_on_sc=False),
      scratch_types=[
          pltpu.VMEM((K_EDGE, CH), jnp.int32),
          pltpu.VMEM((CH, d), jnp.float32),
          pltpu.VMEM_SHARED((n_acc, d), jnp.float32),
          pltpu.SemaphoreType.DMA,
          pltpu.SemaphoreType.DMA,
          pltpu.SemaphoreType.DMA,
          pltpu.SemaphoreType.DMA,
      ],
  )


def _dinv_from_deg(deg_ref):
  return lax.rsqrt(deg_ref[0, :N_NODES, 0:1] + 1.0)


def _tc_a(x_ref, w_ref, deg_ref, g1_ref):
  dinv = _dinv_from_deg(deg_ref)
  t = jnp.dot(x_ref[...], w_ref[...], preferred_element_type=jnp.float32)
  g1_ref[:N_NODES, :] = t * dinv
  g1_ref[N_NODES:, :] = jnp.zeros((N_ACC - N_NODES, 64), jnp.float32)


def _tc_b(s1_ref, g1_ref, deg_ref, b1_ref, ga1_ref, be1_ref, g2_ref):
  dinv = _dinv_from_deg(deg_ref)
  u = dinv * (s1_ref[0, :N_NODES, :] + g1_ref[:N_NODES, :]) + b1_ref[...]
  t = u * (BN_SCALE * ga1_ref[...]) + be1_ref[...]
  h1 = jnp.where(t >= 0, t, 0.02 * t)
  g2_ref[:N_NODES, :] = h1 * dinv
  g2_ref[N_NODES:, :] = jnp.zeros((N_ACC - N_NODES, 64), jnp.float32)


def _tc_c(s2_ref, g2_ref, deg_ref, w2_ref, b2_ref, ga2_ref, be2_ref,
          h2_ref, g3a_ref, g3b_ref):
  dinv = _dinv_from_deg(deg_ref)
  v = dinv * (s2_ref[0, :N_NODES, :] + g2_ref[:N_NODES, :])
  h2 = (jnp.dot(v, w2_ref[...], preferred_element_type=jnp.float32)
        + b2_ref[...]) * (BN_SCALE * ga2_ref[...]) + be2_ref[...]
  h2_ref[...] = h2
  g3 = h2 * dinv
  g3a_ref[:N_NODES, :] = g3[:, :64]
  g3b_ref[:N_NODES, :] = g3[:, 64:]
  g3a_ref[N_NODES:, :] = jnp.zeros((N_ACC - N_NODES, 64), jnp.float32)
  g3b_ref[N_NODES:, :] = jnp.zeros((N_ACC - N_NODES, 64), jnp.float32)


def _tc_d(s3a_ref, s3b_ref, g3a_ref, g3b_ref, deg_ref, w3_ref, b3_ref,
          emb_ref):
  dinv = lax.rsqrt(deg_ref[0, :, 0:1] + 1.0)
  wa = s3a_ref[0] + g3a_ref[...]
  wb = s3b_ref[0] + g3b_ref[...]
  w = dinv * jnp.concatenate([wa, wb], axis=1)
  emb_ref[...] = jnp.dot(w, w3_ref[...],
                         preferred_element_type=jnp.float32) + b3_ref[...]


def _tc_pool(nblk):
  def body(binds_ref, h2_ref, rep_ref, cnt_s):
    i = pl.program_id(0)
    gids = lax.broadcasted_iota(jnp.int32, (1, N_GRAPHS), 1)
    oh = (binds_ref[...] == gids).astype(jnp.float32)
    bs = lax.dot_general(oh, h2_ref[...], (((0,), (0,)), ((), ())),
                         preferred_element_type=jnp.float32)
    bc = jnp.sum(oh, axis=0)[:, None]

    @pl.when(i == 0)
    def _():
      rep_ref[...] = bs
      cnt_s[...] = bc

    @pl.when(i > 0)
    def _():
      rep_ref[...] += bs
      cnt_s[...] += bc

    @pl.when(i == nblk - 1)
    def _():
      rep_ref[...] = rep_ref[...] / jnp.maximum(cnt_s[...], 1.0)

  return body


def _pad_chunks(a, fill):
  total = NS * K_EDGE * CH
  a = jnp.concatenate(
      [a, jnp.full((total - a.shape[0],), fill, dtype=jnp.int32)])
  return a.reshape(NS, K_EDGE, CH)


@jax.jit
def kernel(x, edge_index, binds, W1, b1, g1, be1, W2, b2, g2, be2, W3, b3):
  f32 = jnp.float32
  src = _pad_chunks(edge_index[0].astype(jnp.int32), 0)
  dst = _pad_chunks(edge_index[1].astype(jnp.int32), N_NODES)

  ones_h = jnp.ones((CH, 8), f32)
  zeros_h = jnp.zeros((N_ACC, 8), f32)

  deg_k = _make_hist(8, N_ACC)
  scat64 = _make_edge_scatter(64, N_ACC)

  deg = deg_k(ones_h, dst, zeros_h)

  g1v = pl.pallas_call(
      _tc_a, out_shape=jax.ShapeDtypeStruct((N_ACC, 64), f32),
  )(x, W1, deg)

  S1 = scat64(g1v, src, dst)

  g2v = pl.pallas_call(
      _tc_b, out_shape=jax.ShapeDtypeStruct((N_ACC, 64), f32),
  )(S1, g1v, deg, b1.reshape(1, 64), g1.reshape(1, 64), be1.reshape(1, 64))

  S2 = scat64(g2v, src, dst)

  h2, g3a, g3b = pl.pallas_call(
      _tc_c, out_shape=(jax.ShapeDtypeStruct((N_NODES, 128), f32),
                        jax.ShapeDtypeStruct((N_ACC, 64), f32),
                        jax.ShapeDtypeStruct((N_ACC, 64), f32)),
  )(S2, g2v, deg, W2, b2.reshape(1, 128), g2.reshape(1, 128),
    be2.reshape(1, 128))

  S3a = scat64(g3a, src, dst)
  S3b = scat64(g3b, src, dst)

  B = 2000
  x_emb = pl.pallas_call(
      _tc_d,
      grid=(N_NODES // B,),
      in_specs=[
          pl.BlockSpec((1, B, 64), lambda i: (0, i, 0)),
          pl.BlockSpec((1, B, 64), lambda i: (0, i, 0)),
          pl.BlockSpec((B, 64), lambda i: (i, 0)),
          pl.BlockSpec((B, 64), lambda i: (i, 0)),
          pl.BlockSpec((1, B, 8), lambda i: (0, i, 0)),
          pl.BlockSpec((128, 256), lambda i: (0, 0)),
          pl.BlockSpec((1, 256), lambda i: (0, 0)),
      ],
      out_specs=pl.BlockSpec((B, 256), lambda i: (i, 0)),
      out_shape=jax.ShapeDtypeStruct((N_NODES, 256), f32),
  )(S3a, S3b, g3a, g3b, deg, W3, b3.reshape(1, 256))

  PB = 2000
  x_rep = pl.pallas_call(
      _tc_pool(N_NODES // PB),
      grid=(N_NODES // PB,),
      in_specs=[
          pl.BlockSpec((PB, 1), lambda i: (i, 0)),
          pl.BlockSpec((PB, 128), lambda i: (i, 0)),
      ],
      out_specs=pl.BlockSpec((N_GRAPHS, 128), lambda i: (0, 0)),
      out_shape=jax.ShapeDtypeStruct((N_GRAPHS, 128), f32),
      scratch_shapes=[pltpu.VMEM((N_GRAPHS, 1), f32)],
  )(binds.astype(jnp.int32).reshape(N_NODES, 1), h2)

  return (x_rep, x_emb)


# speedup vs baseline: 1.1274x; 1.1274x over previous
import jax
import jax.numpy as jnp
from jax import lax
from jax.experimental import pallas as pl
from jax.experimental.pallas import tpu as pltpu
from jax.experimental.pallas import tpu_sc as plsc

N_NODES = 10000
N_GRAPHS = 500
BN_EPS = 1e-5
BN_SCALE = (1.0 + BN_EPS) ** -0.5

NC = 2
NS = 16
NW = NC * NS
CH = 128

K_EDGE = 80
N_ACC = 10112


def _make_edge_scatter(d, n_acc):
  mesh = plsc.VectorSubcoreMesh(core_axis_name="c", subcore_axis_name="s")
  z_per = n_acc // NS
  assert K_EDGE % 4 == 0 and z_per % 8 == 0

  def body(table_hbm, src_hbm, dst_hbm, zeros_hbm, out_hbm,
           src_v, dst_v, b0, b1, b2, b3, acc,
           g0, g1, g2, g3, s0, s1, s2, s3):
    c = lax.axis_index("c")
    s = lax.axis_index("s")
    wid = s * NC + c
    bufs = (b0, b1, b2, b3)
    gsems = (g0, g1, g2, g3)
    ssems = (s0, s1, s2, s3)

    pltpu.sync_copy(zeros_hbm.at[pl.ds(s * z_per, z_per)],
                    acc.at[pl.ds(s * z_per, z_per)])
    pltpu.sync_copy(src_hbm.at[wid], src_v)
    pltpu.sync_copy(dst_hbm.at[wid], dst_v)
    plsc.subcore_barrier()

    def gather(k, j):
      pltpu.async_copy(table_hbm.at[src_v.at[k]], bufs[j], gsems[j])

    def scatter(k, j):
      pltpu.async_copy(bufs[j], acc.at[dst_v.at[k]], ssems[j], add=True)

    def wait_gather(k, j):
      pltpu.make_async_copy(table_hbm.at[src_v.at[k]], bufs[j],
                            gsems[j]).wait()

    def wait_scatter(k, j):
      pltpu.make_async_copy(bufs[j], acc.at[dst_v.at[k]], ssems[j]).wait()

    gather(0, 0)
    gather(1, 1)

    def step(m, carry):
      base = 4 * m
      for j in range(4):
        k = base + j
        jn = (j + 2) % 4

        @pl.when(k - 2 >= 0)
        def _():
          wait_scatter(k - 2, jn)

        @pl.when(k + 2 < K_EDGE)
        def _():
          gather(k + 2, jn)

        wait_gather(k, j)
        scatter(k, j)
      return carry

    lax.fori_loop(0, K_EDGE // 4, step, 0)
    wait_scatter(K_EDGE - 2, 2)
    wait_scatter(K_EDGE - 1, 3)
    plsc.subcore_barrier()

    pltpu.sync_copy(acc.at[pl.ds(s * z_per, z_per)],
                    out_hbm.at[c, pl.ds(s * z_per, z_per)])

  return pl.kernel(
      body,
      out_type=jax.ShapeDtypeStruct((NC, n_acc, d), jnp.float32),
      mesh=mesh,
      compiler_params=pltpu.CompilerParams(use_tc_tiling_on_sc=False),
      scratch_types=[
          pltpu.VMEM((K_EDGE, CH), jnp.int32),
          pltpu.VMEM((K_EDGE, CH), jnp.int32),
          pltpu.VMEM((CH, d), jnp.float32),
          pltpu.VMEM((CH, d), jnp.float32),
          pltpu.VMEM((CH, d), jnp.float32),
          pltpu.VMEM((CH, d), jnp.float32),
          pltpu.VMEM_SHARED((n_acc, d), jnp.float32),
          pltpu.SemaphoreType.DMA,
          pltpu.SemaphoreType.DMA,
          pltpu.SemaphoreType.DMA,
          pltpu.SemaphoreType.DMA,
          pltpu.SemaphoreType.DMA,
          pltpu.SemaphoreType.DMA,
          pltpu.SemaphoreType.DMA,
          pltpu.SemaphoreType.DMA,
      ],
  )


def _make_hist(d, n_acc):
  mesh = plsc.VectorSubcoreMesh(core_axis_name="c", subcore_axis_name="s")
  z_per = n_acc // NS
  assert z_per % 8 == 0

  def body(ones_hbm, dst_hbm, zeros_hbm, out_hbm,
           dst_v, rows_v, acc, s0, s1, s2, s3):
    c = lax.axis_index("c")
    s = lax.axis_index("s")
    wid = s * NC + c
    ssems = (s0, s1, s2, s3)

    pltpu.sync_copy(zeros_hbm.at[pl.ds(s * z_per, z_per)],
                    acc.at[pl.ds(s * z_per, z_per)])
    pltpu.sync_copy(dst_hbm.at[wid], dst_v)
    pltpu.sync_copy(ones_hbm, rows_v)
    plsc.subcore_barrier()

    def scatter(k, jj):
      pltpu.async_copy(rows_v, acc.at[dst_v.at[k]], ssems[jj], add=True)

    def wait_scatter(k, jj):
      pltpu.make_async_copy(rows_v, acc.at[dst_v.at[k]], ssems[jj]).wait()

    scatter(0, 0)
    scatter(1, 1)
    scatter(2, 2)

    def step(m, carry):
      base = 4 * m
      for j in range(4):
        k = base + j

        @pl.when(k + 3 < K_EDGE)
        def _():
          scatter(k + 3, (j + 3) % 4)

        wait_scatter(k, j)
      return carry

    lax.fori_loop(0, K_EDGE // 4, step, 0)
    plsc.subcore_barrier()
    pltpu.sync_copy(acc.at[pl.ds(s * z_per, z_per)],
                    out_hbm.at[c, pl.ds(s * z_per, z_per)])

  return pl.kernel(
      body,
      out_type=jax.ShapeDtypeStruct((NC, n_acc, d), jnp.float32),
      mesh=mesh,
      compiler_params=pltpu.CompilerParams(use_tc_tiling_on_sc=False),
      scratch_types=[
          pltpu.VMEM((K_EDGE, CH), jnp.int32),
          pltpu.VMEM((CH, d), jnp.float32),
          pltpu.VMEM_SHARED((n_acc, d), jnp.float32),
          pltpu.SemaphoreType.DMA,
          pltpu.SemaphoreType.DMA,
          pltpu.SemaphoreType.DMA,
          pltpu.SemaphoreType.DMA,
      ],
  )


def _dinv_from_deg(deg_ref):
  cnt = deg_ref[0, :N_NODES, 0:1] + deg_ref[1, :N_NODES, 0:1]
  return lax.rsqrt(cnt + 1.0)


def _tc_a(x_ref, w_ref, deg_ref, g1_ref):
  dinv = _dinv_from_deg(deg_ref)
  t = jnp.dot(x_ref[...], w_ref[...], preferred_element_type=jnp.float32)
  g1_ref[:N_NODES, :] = t * dinv
  g1_ref[N_NODES:, :] = jnp.zeros((N_ACC - N_NODES, 64), jnp.float32)


def _tc_b(s1_ref, g1_ref, deg_ref, b1_ref, ga1_ref, be1_ref, g2_ref):
  dinv = _dinv_from_deg(deg_ref)
  u = dinv * (s1_ref[0, :N_NODES, :] + s1_ref[1, :N_NODES, :]
              + g1_ref[:N_NODES, :]) + b1_ref[...]
  t = u * (BN_SCALE * ga1_ref[...]) + be1_ref[...]
  h1 = jnp.where(t >= 0, t, 0.02 * t)
  g2_ref[:N_NODES, :] = h1 * dinv
  g2_ref[N_NODES:, :] = jnp.zeros((N_ACC - N_NODES, 64), jnp.float32)


def _tc_c(s2_ref, g2_ref, deg_ref, w2_ref, b2_ref, ga2_ref, be2_ref,
          h2_ref, g3a_ref, g3b_ref):
  dinv = _dinv_from_deg(deg_ref)
  v = dinv * (s2_ref[0, :N_NODES, :] + s2_ref[1, :N_NODES, :]
              + g2_ref[:N_NODES, :])
  h2 = (jnp.dot(v, w2_ref[...], preferred_element_type=jnp.float32)
        + b2_ref[...]) * (BN_SCALE * ga2_ref[...]) + be2_ref[...]
  h2_ref[...] = h2
  g3 = h2 * dinv
  g3a_ref[:N_NODES, :] = g3[:, :64]
  g3b_ref[:N_NODES, :] = g3[:, 64:]
  g3a_ref[N_NODES:, :] = jnp.zeros((N_ACC - N_NODES, 64), jnp.float32)
  g3b_ref[N_NODES:, :] = jnp.zeros((N_ACC - N_NODES, 64), jnp.float32)


def _tc_d(s3a_ref, s3b_ref, g3a_ref, g3b_ref, deg_ref, w3_ref, b3_ref,
          emb_ref):
  cnt = deg_ref[0, :, 0:1] + deg_ref[1, :, 0:1]
  dinv = lax.rsqrt(cnt + 1.0)
  wa = s3a_ref[0] + s3a_ref[1] + g3a_ref[...]
  wb = s3b_ref[0] + s3b_ref[1] + g3b_ref[...]
  w = dinv * jnp.concatenate([wa, wb], axis=1)
  emb_ref[...] = jnp.dot(w, w3_ref[...],
                         preferred_element_type=jnp.float32) + b3_ref[...]


def _tc_pool(nblk):
  def body(binds_ref, h2_ref, rep_ref, cnt_s):
    i = pl.program_id(0)
    gids = lax.broadcasted_iota(jnp.int32, (1, N_GRAPHS), 1)
    oh = (binds_ref[...] == gids).astype(jnp.float32)
    bs = lax.dot_general(oh, h2_ref[...], (((0,), (0,)), ((), ())),
                         preferred_element_type=jnp.float32)
    bc = jnp.sum(oh, axis=0)[:, None]

    @pl.when(i == 0)
    def _():
      rep_ref[...] = bs
      cnt_s[...] = bc

    @pl.when(i > 0)
    def _():
      rep_ref[...] += bs
      cnt_s[...] += bc

    @pl.when(i == nblk - 1)
    def _():
      rep_ref[...] = rep_ref[...] / jnp.maximum(cnt_s[...], 1.0)

  return body


def _pad_chunks(a, fill):
  total = NW * K_EDGE * CH
  a = jnp.concatenate(
      [a, jnp.full((total - a.shape[0],), fill, dtype=jnp.int32)])
  return a.reshape(NW, K_EDGE, CH)


@jax.jit
def kernel(x, edge_index, binds, W1, b1, g1, be1, W2, b2, g2, be2, W3, b3):
  f32 = jnp.float32
  src = _pad_chunks(edge_index[0].astype(jnp.int32), 0)
  dst = _pad_chunks(edge_index[1].astype(jnp.int32), N_NODES)

  ones_h = jnp.ones((CH, 8), f32)
  zeros64 = jnp.zeros((N_ACC, 64), f32)
  zeros_h = jnp.zeros((N_ACC, 8), f32)

  deg_k = _make_hist(8, N_ACC)
  scat64 = _make_edge_scatter(64, N_ACC)

  deg = deg_k(ones_h, dst, zeros_h)

  g1v = pl.pallas_call(
      _tc_a, out_shape=jax.ShapeDtypeStruct((N_ACC, 64), f32),
  )(x, W1, deg)

  S1 = scat64(g1v, src, dst, zeros64)

  g2v = pl.pallas_call(
      _tc_b, out_shape=jax.ShapeDtypeStruct((N_ACC, 64), f32),
  )(S1, g1v, deg, b1.reshape(1, 64), g1.reshape(1, 64), be1.reshape(1, 64))

  S2 = scat64(g2v, src, dst, zeros64)

  h2, g3a, g3b = pl.pallas_call(
      _tc_c, out_shape=(jax.ShapeDtypeStruct((N_NODES, 128), f32),
                        jax.ShapeDtypeStruct((N_ACC, 64), f32),
                        jax.ShapeDtypeStruct((N_ACC, 64), f32)),
  )(S2, g2v, deg, W2, b2.reshape(1, 128), g2.reshape(1, 128),
    be2.reshape(1, 128))

  S3a = scat64(g3a, src, dst, zeros64)
  S3b = scat64(g3b, src, dst, zeros64)

  B = 2000
  x_emb = pl.pallas_call(
      _tc_d,
      grid=(N_NODES // B,),
      in_specs=[
          pl.BlockSpec((2, B, 64), lambda i: (0, i, 0)),
          pl.BlockSpec((2, B, 64), lambda i: (0, i, 0)),
          pl.BlockSpec((B, 64), lambda i: (i, 0)),
          pl.BlockSpec((B, 64), lambda i: (i, 0)),
          pl.BlockSpec((2, B, 8), lambda i: (0, i, 0)),
          pl.BlockSpec((128, 256), lambda i: (0, 0)),
          pl.BlockSpec((1, 256), lambda i: (0, 0)),
      ],
      out_specs=pl.BlockSpec((B, 256), lambda i: (i, 0)),
      out_shape=jax.ShapeDtypeStruct((N_NODES, 256), f32),
  )(S3a, S3b, g3a, g3b, deg, W3, b3.reshape(1, 256))

  PB = 2000
  x_rep = pl.pallas_call(
      _tc_pool(N_NODES // PB),
      grid=(N_NODES // PB,),
      in_specs=[
          pl.BlockSpec((PB, 1), lambda i: (i, 0)),
          pl.BlockSpec((PB, 128), lambda i: (i, 0)),
      ],
      out_specs=pl.BlockSpec((N_GRAPHS, 128), lambda i: (0, 0)),
      out_shape=jax.ShapeDtypeStruct((N_GRAPHS, 128), f32),
      scratch_shapes=[pltpu.VMEM((N_GRAPHS, 1), f32)],
  )(binds.astype(jnp.int32).reshape(N_NODES, 1), h2)

  return (x_rep, x_emb)
